# baseline (speedup 1.0000x reference)
import functools

import jax
import jax.numpy as jnp
from jax import lax
from jax.experimental import pallas as pl
from jax.experimental.pallas import tpu as pltpu
from jax.experimental.pallas import tpu_sc as plsc

_NC = 2
_NS = 16
_NW = _NC * _NS
_C = 32


def _sc_body(S, C, x_hbm, pe_hbm, out_hbm, xbuf, pebuf, sem_x, sem_pe):
    D = x_hbm.shape[1]
    rows_per_w = x_hbm.shape[0] // _NW
    n_chunks = rows_per_w // C
    wid = lax.axis_index("s") * _NC + lax.axis_index("c")
    base = wid * rows_per_w
    pe_base = base % S

    def chunk_body(k, _):
        row = base + k * C
        perow = pe_base + k * C
        cx = pltpu.async_copy(x_hbm.at[pl.ds(row, C)], xbuf, sem_x)
        cp = pltpu.async_copy(pe_hbm.at[pl.ds(perow, C)], pebuf, sem_pe)
        cx.wait()
        cp.wait()

        def add_row(i, _):
            def add_vec(j, _):
                s = j * 16
                xbuf[i, pl.ds(s, 16)] = xbuf[i, pl.ds(s, 16)] + pebuf[i, pl.ds(s, 16)]
                return 0

            return lax.fori_loop(0, D // 16, add_vec, 0)

        lax.fori_loop(0, C, add_row, 0)
        pltpu.sync_copy(xbuf, out_hbm.at[pl.ds(row, C)])
        return 0

    lax.fori_loop(0, n_chunks, chunk_body, 0)


def _sc_forward(x, pe):
    B, S, D = x.shape
    x2 = x.reshape(B * S, D)
    mesh = plsc.VectorSubcoreMesh(core_axis_name="c", subcore_axis_name="s")
    out = pl.kernel(
        functools.partial(_sc_body, S, _C),
        out_type=jax.ShapeDtypeStruct((B * S, D), x.dtype),
        mesh=mesh,
        scratch_types=[
            pltpu.VMEM((_C, D), jnp.float32),
            pltpu.VMEM((_C, D), jnp.float32),
            pltpu.SemaphoreType.DMA,
            pltpu.SemaphoreType.DMA,
        ],
    )(x2, pe[:S])
    return out.reshape(B, S, D)


def kernel(x, pe):
    return _sc_forward(x, pe)


# speedup vs baseline: 2.5286x; 2.5286x over previous
import functools

import jax
import jax.numpy as jnp
from jax import lax
from jax.experimental import pallas as pl
from jax.experimental.pallas import tpu as pltpu
from jax.experimental.pallas import tpu_sc as plsc

_NC = 2
_NS = 16
_NW = _NC * _NS
_C = 8


def _sc_body(S, C, x_hbm, pe_hbm, out_hbm,
             xb0, xb1, xb2, xb3, xb4, xb5, xb6, xb7,
             pb0, pb1, sx0, sx1, spe0, spe1, so0, so1):
    D = x_hbm.shape[1]
    B = x_hbm.shape[0] // S
    SEG = D // 16
    SW = S // _NW
    NCH = SW // C

    xb = ((xb0, xb1, xb2, xb3), (xb4, xb5, xb6, xb7))
    pb = (pb0, pb1)
    sx = (sx0, sx1)
    spe = (spe0, spe1)
    so = (so0, so1)

    wid = lax.axis_index("s") * _NC + lax.axis_index("c")
    seq_base = wid * SW

    def start_chunk(c, g):
        row = seq_base + c * C
        pltpu.async_copy(pe_hbm.at[pl.ds(row, C)], pb[g], spe[g])
        for bt in range(B):
            pltpu.async_copy(x_hbm.at[pl.ds(bt * S + row, C)], xb[g][bt], sx[g])

    def wait_in(g):
        pltpu.make_async_copy(pe_hbm.at[pl.ds(0, C)], pb[g], spe[g]).wait()
        for bt in range(B):
            pltpu.make_async_copy(x_hbm.at[pl.ds(0, C)], xb[g][bt], sx[g]).wait()

    def wait_out(g):
        for bt in range(B):
            pltpu.make_async_copy(xb[g][bt], out_hbm.at[pl.ds(0, C)], so[g]).wait()

    def compute(g):
        def row_body(i, _):
            for j in range(SEG):
                sl = pl.ds(j * 16, 16)
                pev = pb[g][i, sl]
                for bt in range(B):
                    xb[g][bt][i, sl] = xb[g][bt][i, sl] + pev
            return 0

        lax.fori_loop(0, C, row_body, 0)

    def store_chunk(c, g):
        row = seq_base + c * C
        for bt in range(B):
            pltpu.async_copy(xb[g][bt], out_hbm.at[pl.ds(bt * S + row, C)], so[g])

    start_chunk(0, 0)

    def kblock(k, _):
        for pc in (0, 1):
            c = 2 * k + pc
            g, og = pc, 1 - pc

            @pl.when(c + 1 < NCH)
            def _prefetch():
                @pl.when(c >= 1)
                def _drain():
                    wait_out(og)

                start_chunk(c + 1, og)

            wait_in(g)
            compute(g)
            store_chunk(c, g)
        return 0

    lax.fori_loop(0, NCH // 2, kblock, 0)
    wait_out(0)
    wait_out(1)


def _sc_forward(x, pe):
    B, S, D = x.shape
    x2 = x.reshape(B * S, D)
    mesh = plsc.VectorSubcoreMesh(core_axis_name="c", subcore_axis_name="s")
    out = pl.kernel(
        functools.partial(_sc_body, S, _C),
        out_type=jax.ShapeDtypeStruct((B * S, D), x.dtype),
        mesh=mesh,
        scratch_types=(
            [pltpu.VMEM((_C, D), jnp.float32) for _ in range(2 * B)]
            + [pltpu.VMEM((_C, D), jnp.float32) for _ in range(2)]
            + [pltpu.SemaphoreType.DMA for _ in range(6)]
        ),
    )(x2, pe[:S])
    return out.reshape(B, S, D)


def kernel(x, pe):
    return _sc_forward(x, pe)
